# baseline (speedup 1.0000x reference)
import functools

import jax
import jax.numpy as jnp
from jax import lax
from jax.experimental import pallas as pl
from jax.experimental.pallas import tpu as pltpu
from jax.experimental.pallas import tpu_sc as plsc

EMBEDDING_DIM = 32
NUM_CORES = 2
NUM_SUBCORES = 16
NUM_WORKERS = NUM_CORES * NUM_SUBCORES
BATCH, SEQ = 16384, 50
ROWS_PER_W = BATCH // NUM_WORKERS
SLOT = 64
IDX_ROWS = ROWS_PER_W * SLOT // 128
RPC = 8
N_CHUNKS = ROWS_PER_W // RPC
NBUF = 6
DEPTH = 4

_mesh = plsc.VectorSubcoreMesh(core_axis_name="c", subcore_axis_name="s")


@functools.partial(
    pl.kernel,
    mesh=_mesh,
    out_type=jax.ShapeDtypeStruct((NUM_WORKERS, IDX_ROWS, 128), jnp.int32),
    scratch_types=[
        pltpu.VMEM((ROWS_PER_W, SEQ), jnp.int32),
        pltpu.VMEM((IDX_ROWS, 128), jnp.int32),
    ],
    compiler_params=pltpu.CompilerParams(
        use_tc_tiling_on_sc=True, needs_layout_passes=False
    ),
)
def _stage_kernel(tok_hbm, idx_out, tok_v, dense_v):
    wid = lax.axis_index("s") * NUM_CORES + lax.axis_index("c")
    row_base = wid * ROWS_PER_W
    pltpu.sync_copy(tok_hbm.at[pl.ds(row_base, ROWS_PER_W)], tok_v)

    lanes = lax.iota(jnp.int32, 16)

    def body(r, carry):
        rows16 = jnp.full((16,), r, jnp.int32)
        drow16 = jnp.full((16,), r // 2, jnp.int32)
        cbase = (r % 2) * SLOT
        for c0 in (0, 16, 32, 34):
            vals = plsc.load_gather(tok_v, [rows16, c0 + lanes])
            plsc.store_scatter(dense_v, [drow16, cbase + c0 + lanes], vals)
        return carry

    lax.fori_loop(0, ROWS_PER_W, body, 0)
    pltpu.sync_copy(dense_v, idx_out.at[wid])


@functools.partial(
    pl.kernel,
    mesh=_mesh,
    out_type=jax.ShapeDtypeStruct((BATCH, SEQ, EMBEDDING_DIM), jnp.float32),
    scratch_types=[
        pltpu.VMEM((IDX_ROWS, 128), jnp.int32),
        pltpu.VMEM((NBUF, RPC, SEQ, EMBEDDING_DIM), jnp.float32),
        pltpu.SemaphoreType.DMA,
        pltpu.SemaphoreType.DMA,
    ],
    compiler_params=pltpu.CompilerParams(use_tc_tiling_on_sc=False),
)
def _gather_kernel(idx_hbm, table_hbm, out_hbm, idx_v, bufs, gsem, wsem):
    wid = lax.axis_index("s") * NUM_CORES + lax.axis_index("c")
    row_base = wid * ROWS_PER_W
    pltpu.sync_copy(idx_hbm.at[wid], idx_v)

    def gather(j, r, b):
        rr = j * RPC + r
        idx = idx_v.at[rr // 2].at[pl.ds((rr % 2) * SLOT, SEQ)]
        return pltpu.make_async_copy(table_hbm.at[idx], bufs.at[b, r], gsem)

    def start_gathers(j, b):
        for r in range(RPC):
            gather(j, r, b).start()

    def wait_gathers(j, b):
        for r in range(RPC):
            gather(j, r, b).wait()

    def write(j, b):
        dst = out_hbm.at[pl.ds(row_base + j * RPC, RPC)]
        return pltpu.make_async_copy(bufs.at[b], dst, wsem)

    for j in range(DEPTH):
        start_gathers(j, j)

    def body(j, carry):
        b = lax.rem(j, NBUF)
        wait_gathers(j, b)
        write(j, b).start()

        @pl.when(j + DEPTH < N_CHUNKS)
        def _():
            jn = j + DEPTH
            bn = lax.rem(jn, NBUF)

            @pl.when(jn >= NBUF)
            def _():
                write(jn - NBUF, bn).wait()

            start_gathers(jn, bn)

        return carry

    lax.fori_loop(0, N_CHUNKS, body, 0)

    for jw in range(N_CHUNKS - NBUF, N_CHUNKS):
        write(jw, jw % NBUF).wait()


def kernel(token_ids, weight):
    idx = _stage_kernel(token_ids)
    return _gather_kernel(idx, weight)
